# Initial kernel scaffold; baseline (speedup 1.0000x reference)
import jax
import jax.numpy as jnp
from jax.experimental import pallas as pl


def kernel(output):
    raise NotImplementedError("write your pallas kernel here")


# speedup vs baseline: 3.5638x; 3.5638x over previous
import jax
import jax.numpy as jnp
from jax.experimental import pallas as pl

_ANCHOR_W = (3.625, 4.875, 11.65625)
_ANCHOR_H = (2.8125, 6.1875, 10.1875)
_NC = 80
_NHW = 19 * 19


def _decode_kernel(in_ref, out_ref):
    i = pl.program_id(0)
    a = i % 3
    v = in_ref[0]

    aw = jnp.where(a == 0, _ANCHOR_W[0], jnp.where(a == 1, _ANCHOR_W[1], _ANCHOR_W[2]))
    ah = jnp.where(a == 0, _ANCHOR_H[0], jnp.where(a == 1, _ANCHOR_H[1], _ANCHOR_H[2]))

    col = jax.lax.broadcasted_iota(jnp.int32, (1, _NHW), 1)
    grid_x = (col % 19).astype(jnp.float32)
    grid_y = (col // 19).astype(jnp.float32)

    bx = jax.nn.sigmoid(v[0:1, :]) + grid_x
    by = jax.nn.sigmoid(v[1:2, :]) + grid_y
    bw = jnp.exp(v[2:3, :]) * aw
    bh = jnp.exp(v[3:4, :]) * ah
    conf = jax.nn.sigmoid(v[4:5, :])

    cls = v[5:, :]
    m = jnp.max(cls, axis=0, keepdims=True)
    e = jnp.exp(cls - m)
    p = e / jnp.sum(e, axis=0, keepdims=True)

    det = jnp.concatenate([bx, by, bw, bh, conf, p], axis=0)
    out_ref[0] = det.T


def kernel(output):
    nB = output.shape[0]
    x = output.reshape(nB * 3, 5 + _NC, _NHW)
    det = pl.pallas_call(
        _decode_kernel,
        grid=(nB * 3,),
        in_specs=[pl.BlockSpec((1, 5 + _NC, _NHW), lambda i: (i, 0, 0))],
        out_specs=pl.BlockSpec((1, _NHW, 5 + _NC), lambda i: (i, 0, 0)),
        out_shape=jax.ShapeDtypeStruct((nB * 3, _NHW, 5 + _NC), jnp.float32),
    )(x)
    return det.reshape(nB, 3, 19, 19, 5 + _NC)


# speedup vs baseline: 7.3644x; 2.0664x over previous
import jax
import jax.numpy as jnp
from jax.experimental import pallas as pl

_ANCHOR_W = (3.625, 4.875, 11.65625)
_ANCHOR_H = (2.8125, 6.1875, 10.1875)
_NC = 80
_NHW = 19 * 19


def _decode_kernel(in_ref, out_ref):
    col = jax.lax.broadcasted_iota(jnp.int32, (1, _NHW), 1)
    grid_x = (col % 19).astype(jnp.float32)
    grid_y = (col // 19).astype(jnp.float32)

    for a in range(3):
        v = in_ref[a]

        bx = jax.nn.sigmoid(v[0:1, :]) + grid_x
        by = jax.nn.sigmoid(v[1:2, :]) + grid_y
        bw = jnp.exp(v[2:3, :]) * _ANCHOR_W[a]
        bh = jnp.exp(v[3:4, :]) * _ANCHOR_H[a]
        conf = jax.nn.sigmoid(v[4:5, :])

        cls = v[5:, :]
        m = jnp.max(cls, axis=0, keepdims=True)
        e = jnp.exp(cls - m)
        p = e / jnp.sum(e, axis=0, keepdims=True)

        det = jnp.concatenate([bx, by, bw, bh, conf, p], axis=0)
        out_ref[0, a] = det.T.reshape(19, 19, 5 + _NC)


def kernel(output):
    nB = output.shape[0]
    x = output.reshape(nB * 3, 5 + _NC, _NHW)
    det = pl.pallas_call(
        _decode_kernel,
        grid=(nB,),
        in_specs=[pl.BlockSpec((3, 5 + _NC, _NHW), lambda i: (i, 0, 0))],
        out_specs=pl.BlockSpec((1, 3, 19, 19, 5 + _NC), lambda i: (i, 0, 0, 0, 0)),
        out_shape=jax.ShapeDtypeStruct((nB, 3, 19, 19, 5 + _NC), jnp.float32),
    )(x)
    return det


# speedup vs baseline: 8.1349x; 1.1046x over previous
import jax
import jax.numpy as jnp
from jax.experimental import pallas as pl

_ANCHOR_W = (3.625, 4.875, 11.65625)
_ANCHOR_H = (2.8125, 6.1875, 10.1875)
_NC = 80
_NHW = 19 * 19


def _decode_kernel(in_ref, out_ref):
    col = jax.lax.broadcasted_iota(jnp.int32, (1, _NHW), 1)
    grid_x = (col % 19).astype(jnp.float32)
    grid_y = (col // 19).astype(jnp.float32)

    for s in range(in_ref.shape[0]):
        a = s % 3
        v = in_ref[s]

        bx = jax.nn.sigmoid(v[0:1, :]) + grid_x
        by = jax.nn.sigmoid(v[1:2, :]) + grid_y
        bw = jnp.exp(v[2:3, :]) * _ANCHOR_W[a]
        bh = jnp.exp(v[3:4, :]) * _ANCHOR_H[a]
        conf = jax.nn.sigmoid(v[4:5, :])

        cls = v[5:, :]
        m = jnp.max(cls, axis=0, keepdims=True)
        e = jnp.exp(cls - m)
        p = e / jnp.sum(e, axis=0, keepdims=True)

        det = jnp.concatenate([bx, by, bw, bh, conf, p], axis=0)
        out_ref[s // 3, a] = det.T.reshape(19, 19, 5 + _NC)


def kernel(output):
    nB = output.shape[0]
    x = output.reshape(nB * 3, 5 + _NC, _NHW)
    bblk = 4
    det = pl.pallas_call(
        _decode_kernel,
        grid=(nB // bblk,),
        in_specs=[pl.BlockSpec((3 * bblk, 5 + _NC, _NHW), lambda i: (i, 0, 0))],
        out_specs=pl.BlockSpec((bblk, 3, 19, 19, 5 + _NC), lambda i: (i, 0, 0, 0, 0)),
        out_shape=jax.ShapeDtypeStruct((nB, 3, 19, 19, 5 + _NC), jnp.float32),
    )(x)
    return det


# speedup vs baseline: 9.0142x; 1.1081x over previous
import jax
import jax.numpy as jnp
from jax.experimental import pallas as pl

_ANCHOR_W = (3.625, 4.875, 11.65625)
_ANCHOR_H = (2.8125, 6.1875, 10.1875)
_NC = 80
_NHW = 19 * 19


def _decode_kernel(in_ref, out_ref):
    col = jax.lax.broadcasted_iota(jnp.int32, (1, _NHW), 1)
    grid_x = (col % 19).astype(jnp.float32)
    grid_y = (col // 19).astype(jnp.float32)

    for s in range(in_ref.shape[0]):
        a = s % 3
        v = in_ref[s]

        bx = jax.nn.sigmoid(v[0:1, :]) + grid_x
        by = jax.nn.sigmoid(v[1:2, :]) + grid_y
        bw = jnp.exp(v[2:3, :]) * _ANCHOR_W[a]
        bh = jnp.exp(v[3:4, :]) * _ANCHOR_H[a]
        conf = jax.nn.sigmoid(v[4:5, :])

        cls = v[5:, :]
        m = jnp.max(cls, axis=0, keepdims=True)
        e = jnp.exp(cls - m)
        p = e / jnp.sum(e, axis=0, keepdims=True)

        det = jnp.concatenate([bx, by, bw, bh, conf, p], axis=0)
        det_t = det.T
        for h in range(19):
            out_ref[s // 3, a, h] = det_t[19 * h:19 * h + 19, :]


def kernel(output):
    nB = output.shape[0]
    x = output.reshape(nB * 3, 5 + _NC, _NHW)
    bblk = 4
    det = pl.pallas_call(
        _decode_kernel,
        grid=(nB // bblk,),
        in_specs=[pl.BlockSpec((3 * bblk, 5 + _NC, _NHW), lambda i: (i, 0, 0))],
        out_specs=pl.BlockSpec((bblk, 3, 19, 19, 5 + _NC), lambda i: (i, 0, 0, 0, 0)),
        out_shape=jax.ShapeDtypeStruct((nB, 3, 19, 19, 5 + _NC), jnp.float32),
    )(x)
    return det


# speedup vs baseline: 9.2828x; 1.0298x over previous
import jax
import jax.numpy as jnp
from jax.experimental import pallas as pl

_ANCHOR_W = (3.625, 4.875, 11.65625)
_ANCHOR_H = (2.8125, 6.1875, 10.1875)
_NC = 80
_NHW = 19 * 19


def _decode_kernel(in_ref, out_ref):
    col = jax.lax.broadcasted_iota(jnp.int32, (1, _NHW), 1)
    grid_x = (col % 19).astype(jnp.float32)
    grid_y = (col // 19).astype(jnp.float32)

    for s in range(in_ref.shape[0]):
        a = s % 3
        v = in_ref[s]

        bx = jax.nn.sigmoid(v[0:1, :]) + grid_x
        by = jax.nn.sigmoid(v[1:2, :]) + grid_y
        bw = jnp.exp(v[2:3, :]) * _ANCHOR_W[a]
        bh = jnp.exp(v[3:4, :]) * _ANCHOR_H[a]
        conf = jax.nn.sigmoid(v[4:5, :])

        cls = v[5:, :]
        m = jnp.max(cls, axis=0, keepdims=True)
        e = jnp.exp(cls - m)
        p = e / jnp.sum(e, axis=0, keepdims=True)

        det = jnp.concatenate([bx, by, bw, bh, conf, p], axis=0)
        det_t = det.T
        for h in range(19):
            out_ref[s // 3, a, h] = det_t[19 * h:19 * h + 19, :]


def kernel(output):
    nB = output.shape[0]
    x = output.reshape(nB * 3, 5 + _NC, _NHW)
    bblk = 8
    det = pl.pallas_call(
        _decode_kernel,
        grid=(nB // bblk,),
        in_specs=[pl.BlockSpec((3 * bblk, 5 + _NC, _NHW), lambda i: (i, 0, 0))],
        out_specs=pl.BlockSpec((bblk, 3, 19, 19, 5 + _NC), lambda i: (i, 0, 0, 0, 0)),
        out_shape=jax.ShapeDtypeStruct((nB, 3, 19, 19, 5 + _NC), jnp.float32),
    )(x)
    return det
